# baseline (speedup 1.0000x reference)
import jax
import jax.numpy as jnp
from jax.experimental import pallas as pl
from jax.experimental.pallas import tpu as pltpu
from jax.experimental.pallas import tpu_sc as plsc

N = 10000
K = 32
D = 128
T = 3
NK = N * K

NB_PRE = 2000
NB_POST = 1000
GW = 128
NK_PAD = 327680


def _pre_body(f_ref, xp_ref, ow1_ref, ob1_ref, ow2_ref, ob2_ref, w3_ref,
              wf_ref, g_ref, cw_ref):
    f = f_ref[...]
    xp = xp_ref[...]
    h = jnp.maximum(
        jnp.dot(f, ow1_ref[...], preferred_element_type=jnp.float32)
        + ob1_ref[...], 0.0)
    delta = jnp.dot(h, ow2_ref[...], preferred_element_type=jnp.float32) \
        + ob2_ref[...]
    center = xp + delta
    w3 = w3_ref[...]
    g_ref[...] = jnp.dot(xp, w3, preferred_element_type=jnp.float32) \
        + jnp.dot(f, wf_ref[...], preferred_element_type=jnp.float32)
    cw_ref[...] = jnp.dot(center, w3, preferred_element_type=jnp.float32)


def _pre(f, xp, ow1, ob1, ow2p, ob2p, w3p, wf):
    full = lambda shape: pl.BlockSpec(shape, lambda i: (0, 0))
    blk = lambda: pl.BlockSpec((NB_PRE, D), lambda i: (i, 0))
    return pl.pallas_call(
        _pre_body,
        grid=(N // NB_PRE,),
        in_specs=[blk(), blk(), full((D, D)), full((1, D)), full((D, D)),
                  full((1, D)), full((D, D)), full((D, D))],
        out_specs=[blk(), blk()],
        out_shape=[jax.ShapeDtypeStruct((N, D), jnp.float32),
                   jax.ShapeDtypeStruct((N, D), jnp.float32)],
    )(f, xp, ow1, ob1, ow2p, ob2p, w3p, wf)


def _sc_gather(table, idx2d):
    @pl.kernel(
        out_type=jax.ShapeDtypeStruct((NK_PAD, D), table.dtype),
        mesh=plsc.VectorSubcoreMesh(core_axis_name="core",
                                    subcore_axis_name="subcore"),
        scratch_types=[pltpu.VMEM_SHARED((N, D), jnp.float32)],
    )
    def gather_kernel(x_hbm, i_hbm, o_hbm, spm):
        @pl.when(jax.lax.axis_index("subcore") == 0)
        def _():
            pltpu.sync_copy(x_hbm, spm)

        plsc.subcore_barrier()

        def body(i_vmem, o_vmem):
            pltpu.sync_copy(spm.at[i_vmem.at[0]], o_vmem)

        nwin = NK_PAD // GW
        per_unit = nwin // 32
        u = jax.lax.axis_index("core") * 16 + jax.lax.axis_index("subcore")
        r0 = u * (per_unit * GW)
        i_slice = i_hbm.at[:, pl.ds(r0, per_unit * GW)]
        o_slice = o_hbm.at[pl.ds(r0, per_unit * GW), :]
        pltpu.emit_pipeline(
            body,
            grid=(per_unit,),
            in_specs=[pl.BlockSpec((1, GW), index_map=lambda j: (0, j))],
            out_specs=[pl.BlockSpec((GW, D), index_map=lambda j: (j, 0))],
        )(i_slice, o_slice)

    return gather_kernel(table, idx2d)


def _post_body(ag_ref, cw_ref, f_ref, eb1_ref, ew2_ref, eb2_ref, uw1a_ref,
               uw1b_ref, ub1_ref, uw2_ref, ub2_ref, o_ref):
    ag = ag_ref[...].reshape(NB_POST, K, D)
    cw = cw_ref[...]
    e = jnp.maximum(ag - cw[:, None, :] + eb1_ref[...][None], 0.0)
    ef = jnp.dot(e.reshape(NB_POST * K, D).astype(jnp.bfloat16),
                 ew2_ref[...].astype(jnp.bfloat16),
                 preferred_element_type=jnp.float32) + eb2_ref[...]
    agg = jnp.max(ef.reshape(NB_POST, K, D), axis=1)
    f = f_ref[...]
    u = jnp.maximum(
        jnp.dot(agg.astype(jnp.bfloat16), uw1a_ref[...].astype(jnp.bfloat16),
                preferred_element_type=jnp.float32)
        + jnp.dot(f.astype(jnp.bfloat16), uw1b_ref[...].astype(jnp.bfloat16),
                  preferred_element_type=jnp.float32)
        + ub1_ref[...], 0.0)
    o_ref[...] = f + jnp.dot(u.astype(jnp.bfloat16),
                             uw2_ref[...].astype(jnp.bfloat16),
                             preferred_element_type=jnp.float32) \
        + ub2_ref[...]


def _post(ag, cw, f, eb1, ew2, eb2, uw1a, uw1b, ub1, uw2, ub2):
    full = lambda shape: pl.BlockSpec(shape, lambda i: (0, 0))
    blk = lambda: pl.BlockSpec((NB_POST, D), lambda i: (i, 0))
    return pl.pallas_call(
        _post_body,
        grid=(N // NB_POST,),
        in_specs=[pl.BlockSpec((NB_POST * K, D), lambda i: (i, 0)),
                  blk(), blk(), full((1, D)), full((D, D)), full((1, D)),
                  full((D, D)), full((D, D)), full((1, D)), full((D, D)),
                  full((1, D))],
        out_specs=blk(),
        out_shape=jax.ShapeDtypeStruct((N, D), jnp.float32),
    )(ag, cw, f, eb1, ew2, eb2, uw1a, uw1b, ub1, uw2, ub2)


def kernel(xyz, features, knn_idx, off_w1, off_b1, off_w2, off_b2,
           edge_w1, edge_b1, edge_w2, edge_b2,
           upd_w1, upd_b1, upd_w2, upd_b2):
    xyz0 = xyz[0]
    feats = features[0]
    idx = knn_idx[0].astype(jnp.int32).reshape(NK)
    idx2d = jnp.pad(idx, (0, NK_PAD - NK)).reshape(1, NK_PAD)

    xp = jnp.pad(xyz0, ((0, 0), (0, D - 3)))
    ow2p = jnp.pad(off_w2, ((0, 0), (0, D - 3)))
    ob2p = jnp.pad(off_b2, (0, D - 3)).reshape(1, D)
    ob1 = off_b1.reshape(1, D)

    for t in range(T):
        w3p = jnp.pad(edge_w1[t, :3, :], ((0, D - 3), (0, 0)))
        wf = edge_w1[t, 3:, :]
        g, cw = _pre(feats, xp, off_w1, ob1, ow2p, ob2p, w3p, wf)
        ag = _sc_gather(g, idx2d)
        feats = _post(ag, cw, feats, edge_b1[t].reshape(1, D), edge_w2[t],
                      edge_b2[t].reshape(1, D), upd_w1[t, :D, :],
                      upd_w1[t, D:, :], upd_b1[t].reshape(1, D), upd_w2[t],
                      upd_b2[t].reshape(1, D))
    return feats[None]


# speedup vs baseline: 1.0068x; 1.0068x over previous
import jax
import jax.numpy as jnp
from jax.experimental import pallas as pl
from jax.experimental.pallas import tpu as pltpu
from jax.experimental.pallas import tpu_sc as plsc

N = 10000
K = 32
D = 128
T = 3
NK = N * K

NB_PRE = 2000
NB_POST = 1000
GW = 128
NK_PAD = 327680


def _pre_body(f_ref, xp_ref, ow1_ref, ob1_ref, ow2_ref, ob2_ref, w3_ref,
              wf_ref, g_ref, cw_ref):
    f = f_ref[...]
    xp = xp_ref[...]
    h = jnp.maximum(
        jnp.dot(f, ow1_ref[...], preferred_element_type=jnp.float32)
        + ob1_ref[...], 0.0)
    delta = jnp.dot(h, ow2_ref[...], preferred_element_type=jnp.float32) \
        + ob2_ref[...]
    center = xp + delta
    w3 = w3_ref[...]
    g_ref[...] = jnp.dot(xp, w3, preferred_element_type=jnp.float32) \
        + jnp.dot(f, wf_ref[...], preferred_element_type=jnp.float32)
    cw_ref[...] = jnp.dot(center, w3, preferred_element_type=jnp.float32)


def _pre(f, xp, ow1, ob1, ow2p, ob2p, w3p, wf):
    full = lambda shape: pl.BlockSpec(shape, lambda i: (0, 0))
    blk = lambda: pl.BlockSpec((NB_PRE, D), lambda i: (i, 0))
    return pl.pallas_call(
        _pre_body,
        grid=(N // NB_PRE,),
        in_specs=[blk(), blk(), full((D, D)), full((1, D)), full((D, D)),
                  full((1, D)), full((D, D)), full((D, D))],
        out_specs=[blk(), blk()],
        out_shape=[jax.ShapeDtypeStruct((N, D), jnp.float32),
                   jax.ShapeDtypeStruct((N, D), jnp.float32)],
    )(f, xp, ow1, ob1, ow2p, ob2p, w3p, wf)


def _sc_gather(table, idx2d):
    @pl.kernel(
        out_type=jax.ShapeDtypeStruct((NK_PAD, D), table.dtype),
        mesh=plsc.VectorSubcoreMesh(core_axis_name="core",
                                    subcore_axis_name="subcore"),
        scratch_types=[pltpu.VMEM_SHARED((N, D), jnp.float32)],
    )
    def gather_kernel(x_hbm, i_hbm, o_hbm, spm):
        @pl.when(jax.lax.axis_index("subcore") == 0)
        def _():
            pltpu.sync_copy(x_hbm, spm)

        plsc.subcore_barrier()

        def body(i_vmem, o_vmem):
            pltpu.sync_copy(spm.at[i_vmem.at[0]], o_vmem)

        nwin = NK_PAD // GW
        per_unit = nwin // 32
        u = jax.lax.axis_index("core") * 16 + jax.lax.axis_index("subcore")
        r0 = u * (per_unit * GW)
        i_slice = i_hbm.at[:, pl.ds(r0, per_unit * GW)]
        o_slice = o_hbm.at[pl.ds(r0, per_unit * GW), :]
        pltpu.emit_pipeline(
            body,
            grid=(per_unit,),
            in_specs=[pl.BlockSpec((1, GW), index_map=lambda j: (0, j))],
            out_specs=[pl.BlockSpec((GW, D), index_map=lambda j: (j, 0))],
        )(i_slice, o_slice)

    return gather_kernel(table, idx2d)


def _post_body(ag_ref, cw_ref, f_ref, eb1_ref, ew2_ref, eb2_ref, uw1a_ref,
               uw1b_ref, ub1_ref, uw2_ref, ub2_ref, o_ref):
    ag = ag_ref[...].reshape(NB_POST, K, D)
    cw = cw_ref[...]
    e = jnp.maximum(ag - cw[:, None, :] + eb1_ref[...][None], 0.0)
    ef = jnp.dot(e.reshape(NB_POST * K, D).astype(jnp.bfloat16),
                 ew2_ref[...].astype(jnp.bfloat16),
                 preferred_element_type=jnp.float32) + eb2_ref[...]
    agg = jnp.max(ef.reshape(NB_POST, K, D), axis=1)
    f = f_ref[...]
    u = jnp.maximum(
        jnp.dot(agg, uw1a_ref[...], preferred_element_type=jnp.float32)
        + jnp.dot(f, uw1b_ref[...], preferred_element_type=jnp.float32)
        + ub1_ref[...], 0.0)
    o_ref[...] = f + jnp.dot(u, uw2_ref[...],
                             preferred_element_type=jnp.float32) \
        + ub2_ref[...]


def _post(ag, cw, f, eb1, ew2, eb2, uw1a, uw1b, ub1, uw2, ub2):
    full = lambda shape: pl.BlockSpec(shape, lambda i: (0, 0))
    blk = lambda: pl.BlockSpec((NB_POST, D), lambda i: (i, 0))
    return pl.pallas_call(
        _post_body,
        grid=(N // NB_POST,),
        in_specs=[pl.BlockSpec((NB_POST * K, D), lambda i: (i, 0)),
                  blk(), blk(), full((1, D)), full((D, D)), full((1, D)),
                  full((D, D)), full((D, D)), full((1, D)), full((D, D)),
                  full((1, D))],
        out_specs=blk(),
        out_shape=jax.ShapeDtypeStruct((N, D), jnp.float32),
    )(ag, cw, f, eb1, ew2, eb2, uw1a, uw1b, ub1, uw2, ub2)


def kernel(xyz, features, knn_idx, off_w1, off_b1, off_w2, off_b2,
           edge_w1, edge_b1, edge_w2, edge_b2,
           upd_w1, upd_b1, upd_w2, upd_b2):
    xyz0 = xyz[0]
    feats = features[0]
    idx = knn_idx[0].astype(jnp.int32).reshape(NK)
    idx2d = jnp.pad(idx, (0, NK_PAD - NK)).reshape(1, NK_PAD)

    xp = jnp.pad(xyz0, ((0, 0), (0, D - 3)))
    ow2p = jnp.pad(off_w2, ((0, 0), (0, D - 3)))
    ob2p = jnp.pad(off_b2, (0, D - 3)).reshape(1, D)
    ob1 = off_b1.reshape(1, D)

    for t in range(T):
        w3p = jnp.pad(edge_w1[t, :3, :], ((0, D - 3), (0, 0)))
        wf = edge_w1[t, 3:, :]
        g, cw = _pre(feats, xp, off_w1, ob1, ow2p, ob2p, w3p, wf)
        ag = _sc_gather(g, idx2d)
        feats = _post(ag, cw, feats, edge_b1[t].reshape(1, D), edge_w2[t],
                      edge_b2[t].reshape(1, D), upd_w1[t, :D, :],
                      upd_w1[t, D:, :], upd_b1[t].reshape(1, D), upd_w2[t],
                      upd_b2[t].reshape(1, D))
    return feats[None]
